# Initial kernel scaffold; baseline (speedup 1.0000x reference)
import jax
import jax.numpy as jnp
from jax.experimental import pallas as pl


def kernel(target_edge_index, x, embed_edge_index, pitch_score, onset_score, params):
    raise NotImplementedError("write your pallas kernel here")


# speedup vs baseline: 1.0530x; 1.0530x over previous
import functools

import jax
import jax.numpy as jnp
from jax import lax
from jax.experimental import pallas as pl
from jax.experimental.pallas import tpu as pltpu

NN = 10000
EE = 320000
ETN = 100000
HD = 256


def _mm_body(x_ref, w_ref, o_ref):
    o_ref[...] = lax.dot_general(
        x_ref[...], w_ref[...], (((1,), (1,)), ((), ())),
        preferred_element_type=jnp.float32)


def _matmul_t(x, w, block_m=2000):
    M, K = x.shape
    H = w.shape[0]
    return pl.pallas_call(
        _mm_body,
        grid=(M // block_m,),
        in_specs=[pl.BlockSpec((block_m, K), lambda i: (i, 0)),
                  pl.BlockSpec((H, K), lambda i: (0, 0))],
        out_specs=pl.BlockSpec((block_m, H), lambda i: (i, 0)),
        out_shape=jax.ShapeDtypeStruct((M, H), jnp.float32),
    )(x, w)


def _post_body(agg_ref, s_ref, cb_ref, g_ref, b_ref, h_ref):
    t = jnp.maximum(agg_ref[...] + s_ref[...] + cb_ref[...], 0.0)
    mu = jnp.mean(t, axis=-1, keepdims=True)
    var = jnp.mean((t - mu) ** 2, axis=-1, keepdims=True)
    h_ref[...] = (t - mu) * lax.rsqrt(var + 1e-5) * g_ref[...] + b_ref[...]


def _post_conv(agg, s, conv_b, ln_g, ln_b, block_m=2000):
    M, H = agg.shape
    vec = pl.BlockSpec((1, H), lambda i: (0, 0))
    return pl.pallas_call(
        _post_body,
        grid=(M // block_m,),
        in_specs=[pl.BlockSpec((block_m, H), lambda i: (i, 0)),
                  pl.BlockSpec((block_m, H), lambda i: (i, 0)),
                  vec, vec, vec],
        out_specs=pl.BlockSpec((block_m, H), lambda i: (i, 0)),
        out_shape=jax.ShapeDtypeStruct((M, H), jnp.float32),
    )(agg, s, conv_b.reshape(1, H), ln_g.reshape(1, H), ln_b.reshape(1, H))


def _jk_body(h1_ref, h2_ref, agg3_ref, s3_ref, cb3_ref,
             w1_ref, w2_ref, w3_ref, bjk_ref, o_ref):
    h3 = agg3_ref[...] + s3_ref[...] + cb3_ref[...]
    acc = lax.dot_general(h1_ref[...], w1_ref[...], (((1,), (1,)), ((), ())),
                          preferred_element_type=jnp.float32)
    acc += lax.dot_general(h2_ref[...], w2_ref[...], (((1,), (1,)), ((), ())),
                           preferred_element_type=jnp.float32)
    acc += lax.dot_general(h3, w3_ref[...], (((1,), (1,)), ((), ())),
                           preferred_element_type=jnp.float32)
    o_ref[...] = acc + bjk_ref[...]


def _jk(h1, h2, agg3, s3, cb3, wjk, bjk, block_m=2000):
    M, H = h1.shape
    w1 = wjk[:, :H]
    w2 = wjk[:, H:2 * H]
    w3 = wjk[:, 2 * H:]
    blk = pl.BlockSpec((block_m, H), lambda i: (i, 0))
    wblk = pl.BlockSpec((H, H), lambda i: (0, 0))
    vec = pl.BlockSpec((1, H), lambda i: (0, 0))
    return pl.pallas_call(
        _jk_body,
        grid=(M // block_m,),
        in_specs=[blk, blk, blk, blk, vec, wblk, wblk, wblk, vec],
        out_specs=blk,
        out_shape=jax.ShapeDtypeStruct((M, H), jnp.float32),
    )(h1, h2, agg3, s3, cb3.reshape(1, H), w1, w2, w3, bjk.reshape(1, H))


def _mlp_body(pa_ref, pb_ref, po_ref, wpo_ref, b1_ref, w2_ref, b2_ref,
              w3_ref, b3_ref, o_ref):
    z1 = pa_ref[...] + pb_ref[...] + b1_ref[...]
    z1 += lax.dot_general(po_ref[...], wpo_ref[...], (((1,), (1,)), ((), ())),
                          preferred_element_type=jnp.float32)
    z1 = jnp.maximum(z1, 0.0)
    z2 = lax.dot_general(z1, w2_ref[...], (((1,), (1,)), ((), ())),
                         preferred_element_type=jnp.float32) + b2_ref[...]
    z2 = jnp.maximum(z2, 0.0)
    z = jnp.sum(z2 * w3_ref[...], axis=-1, keepdims=True) + b3_ref[...]
    o_ref[...] = jax.nn.sigmoid(z)


def _mlp(pa, pb, po, wpo, b1, w2, b2, w3, b3, block_m=2000):
    M, H = pa.shape
    H2 = w2.shape[0]
    return pl.pallas_call(
        _mlp_body,
        grid=(M // block_m,),
        in_specs=[pl.BlockSpec((block_m, H), lambda i: (i, 0)),
                  pl.BlockSpec((block_m, H), lambda i: (i, 0)),
                  pl.BlockSpec((block_m, 4), lambda i: (i, 0)),
                  pl.BlockSpec((H, 4), lambda i: (0, 0)),
                  pl.BlockSpec((1, H), lambda i: (0, 0)),
                  pl.BlockSpec((H2, H), lambda i: (0, 0)),
                  pl.BlockSpec((1, H2), lambda i: (0, 0)),
                  pl.BlockSpec((1, H2), lambda i: (0, 0)),
                  pl.BlockSpec((1, 1), lambda i: (0, 0))],
        out_specs=pl.BlockSpec((block_m, 1), lambda i: (i, 0)),
        out_shape=jax.ShapeDtypeStruct((M, 1), jnp.float32),
    )(pa, pb, po, wpo, b1, w2, b2, w3, b3)


def kernel(target_edge_index, x, embed_edge_index, pitch_score, onset_score,
           params):
    src, dst = embed_edge_index[0], embed_edge_index[1]
    convs = params['convs']

    h = x
    hs = []
    agg3 = None
    s3 = None
    for i in range(3):
        p = convs[i]
        wall = jnp.concatenate([p['Wk'], p['Wq'], p['Wv'], p['Ws']], axis=0)
        kqvs = _matmul_t(h, wall)
        k = kqvs[:, :HD]
        q = kqvs[:, HD:2 * HD]
        v = kqvs[:, 2 * HD:3 * HD]
        s = kqvs[:, 3 * HD:]
        eta = jax.nn.sigmoid(k[dst] + q[src])
        msg = eta * v[src]
        agg = jax.ops.segment_sum(msg, dst, num_segments=NN)
        if i != 2:
            h = _post_conv(agg, s, p['b'], params['ln_g'], params['ln_b'])
            hs.append(h)
        else:
            agg3, s3 = agg, s

    hjk = _jk(hs[0], hs[1], agg3, s3, convs[2]['b'],
              params['Wjk'], params['bjk'])

    wa = params['Wp1'][:, :HD]
    wb = params['Wp1'][:, HD:2 * HD]
    wab = jnp.concatenate([wa, wb], axis=0)
    ab = _matmul_t(hjk, wab)
    a_tab = ab[:, :HD]
    b_tab = ab[:, HD:]

    ts, td = target_edge_index[0], target_edge_index[1]
    pa = a_tab[ts]
    pb = b_tab[td]

    po = jnp.concatenate(
        [pitch_score, onset_score,
         jnp.zeros((ETN, 1), jnp.float32)], axis=1)
    wpo = jnp.concatenate(
        [params['Wp1'][:, 2 * HD:],
         jnp.zeros((HD, 1), jnp.float32)], axis=1)

    return _mlp(pa, pb, po, wpo,
                params['bp1'].reshape(1, HD),
                params['Wp2'],
                params['bp2'].reshape(1, HD // 2),
                params['Wp3'].reshape(1, HD // 2),
                params['bp3'].reshape(1, 1))
